# Initial kernel scaffold; baseline (speedup 1.0000x reference)
import jax
import jax.numpy as jnp
from jax.experimental import pallas as pl


def kernel(x, edge_index, W1, b1, W2, b2, W3, b3):
    raise NotImplementedError("write your pallas kernel here")


# speedup vs baseline: 6.3943x; 6.3943x over previous
import functools
import jax
import jax.numpy as jnp
from jax import lax
from jax.experimental import pallas as pl
from jax.experimental.pallas import tpu as pltpu
from jax.experimental.pallas import tpu_sc as plsc

NN = 10000
FF = 128
EE = 320000

NCORE = 2
NSUB = 16
NWORK = NCORE * NSUB
KCH = 128
CCH = 80
EPAD = NWORK * CCH * KCH
RPT = 632
ACC_ROWS = NSUB * RPT

BLK = 2000

_mesh = plsc.VectorSubcoreMesh(core_axis_name="c", subcore_axis_name="s")


@functools.partial(
    pl.kernel,
    mesh=_mesh,
    out_type=jax.ShapeDtypeStruct((NCORE, ACC_ROWS, FF), jnp.float32),
    scratch_types=[
        pltpu.VMEM((CCH, KCH), jnp.int32),
        pltpu.VMEM((KCH, FF), jnp.float32),
        pltpu.VMEM_SHARED((ACC_ROWS, FF), jnp.float32),
    ],
)
def _sc_deg(dst_hbm, zeros_hbm, ones_hbm, out_hbm, idx_v, ones_v, deg_sh):
    c = lax.axis_index("c")
    s = lax.axis_index("s")
    w = s * NCORE + c
    pltpu.sync_copy(zeros_hbm, deg_sh.at[pl.ds(s * RPT, RPT)])
    pltpu.sync_copy(dst_hbm.at[w], idx_v)
    pltpu.sync_copy(ones_hbm, ones_v)
    plsc.subcore_barrier()

    def body(i, carry):
        pltpu.sync_copy(ones_v, deg_sh.at[idx_v.at[i]], add=True)
        return carry

    lax.fori_loop(0, CCH, body, 0)
    plsc.subcore_barrier()
    pltpu.sync_copy(deg_sh.at[pl.ds(s * RPT, RPT)],
                    out_hbm.at[c, pl.ds(s * RPT, RPT)])


@functools.partial(
    pl.kernel,
    mesh=_mesh,
    out_type=jax.ShapeDtypeStruct((NCORE, ACC_ROWS, FF), jnp.float32),
    scratch_types=[
        pltpu.VMEM((CCH, KCH), jnp.int32),
        pltpu.VMEM((CCH, KCH), jnp.int32),
        pltpu.VMEM((KCH, FF), jnp.float32),
        pltpu.VMEM_SHARED((ACC_ROWS, FF), jnp.float32),
        pltpu.SemaphoreType.DMA,
    ],
)
def _sc_spmm(g_hbm, src_hbm, dst_hbm, zeros_hbm, out_hbm,
             sidx_v, didx_v, rows_v, acc_sh, sem):
    c = lax.axis_index("c")
    s = lax.axis_index("s")
    w = s * NCORE + c
    pltpu.sync_copy(zeros_hbm, acc_sh.at[pl.ds(s * RPT, RPT)])
    pltpu.sync_copy(src_hbm.at[w], sidx_v)
    pltpu.sync_copy(dst_hbm.at[w], didx_v)
    plsc.subcore_barrier()

    def body(i, carry):
        pltpu.async_copy(g_hbm.at[sidx_v.at[i]], rows_v, sem).wait()
        pltpu.sync_copy(rows_v, acc_sh.at[didx_v.at[i]], add=True)
        return carry

    lax.fori_loop(0, CCH, body, 0)
    plsc.subcore_barrier()
    pltpu.sync_copy(acc_sh.at[pl.ds(s * RPT, RPT)],
                    out_hbm.at[c, pl.ds(s * RPT, RPT)])


def _prep_body(parts_ref, dinv_ref):
    deg = parts_ref[0, :NN, :1] + parts_ref[1, :NN, :1] + 1.0
    dinv_ref[...] = lax.rsqrt(deg)


def _tc_prep(deg_parts):
    return pl.pallas_call(
        _prep_body,
        out_shape=jax.ShapeDtypeStruct((NN, 1), jnp.float32),
    )(deg_parts)


def _mm1_body(x_ref, w_ref, dinv_ref, hlin_ref, g_ref):
    hl = jnp.dot(x_ref[...], w_ref[...],
                 preferred_element_type=jnp.float32,
                 precision=lax.Precision.HIGHEST)
    hlin_ref[...] = hl
    g_ref[...] = hl * dinv_ref[...]


def _tc_mm1(x, w, dinv):
    return pl.pallas_call(
        _mm1_body,
        grid=(NN // BLK,),
        in_specs=[
            pl.BlockSpec((BLK, FF), lambda i: (i, 0)),
            pl.BlockSpec((FF, FF), lambda i: (0, 0)),
            pl.BlockSpec((BLK, 1), lambda i: (i, 0)),
        ],
        out_specs=[
            pl.BlockSpec((BLK, FF), lambda i: (i, 0)),
            pl.BlockSpec((BLK, FF), lambda i: (i, 0)),
        ],
        out_shape=[
            jax.ShapeDtypeStruct((NN, FF), jnp.float32),
            jax.ShapeDtypeStruct((NN, FF), jnp.float32),
        ],
    )(x, w, dinv)


def _mid_body(acc_ref, hlin_ref, dinv_ref, b_ref, w_ref, hlinn_ref, gn_ref):
    dv = dinv_ref[...]
    a = acc_ref[0] + acc_ref[1]
    sfull = dv * a + (dv * dv) * hlin_ref[...] + b_ref[...]
    h = jnp.maximum(sfull, 0.0)
    hn = jnp.dot(h, w_ref[...],
                 preferred_element_type=jnp.float32,
                 precision=lax.Precision.HIGHEST)
    hlinn_ref[...] = hn
    gn_ref[...] = hn * dv


def _tc_mid(acc, hlin, dinv, b, w):
    return pl.pallas_call(
        _mid_body,
        grid=(NN // BLK,),
        in_specs=[
            pl.BlockSpec((NCORE, BLK, FF), lambda i: (0, i, 0)),
            pl.BlockSpec((BLK, FF), lambda i: (i, 0)),
            pl.BlockSpec((BLK, 1), lambda i: (i, 0)),
            pl.BlockSpec((1, FF), lambda i: (0, 0)),
            pl.BlockSpec((FF, FF), lambda i: (0, 0)),
        ],
        out_specs=[
            pl.BlockSpec((BLK, FF), lambda i: (i, 0)),
            pl.BlockSpec((BLK, FF), lambda i: (i, 0)),
        ],
        out_shape=[
            jax.ShapeDtypeStruct((NN, FF), jnp.float32),
            jax.ShapeDtypeStruct((NN, FF), jnp.float32),
        ],
    )(acc, hlin, dinv, b, w)


def _final_body(acc_ref, hlin_ref, dinv_ref, b_ref, out_ref):
    dv = dinv_ref[...]
    a = acc_ref[0] + acc_ref[1]
    out_ref[...] = dv * a + (dv * dv) * hlin_ref[...] + b_ref[...]


def _tc_final(acc, hlin, dinv, b):
    return pl.pallas_call(
        _final_body,
        grid=(NN // BLK,),
        in_specs=[
            pl.BlockSpec((NCORE, BLK, FF), lambda i: (0, i, 0)),
            pl.BlockSpec((BLK, FF), lambda i: (i, 0)),
            pl.BlockSpec((BLK, 1), lambda i: (i, 0)),
            pl.BlockSpec((1, FF), lambda i: (0, 0)),
        ],
        out_specs=pl.BlockSpec((BLK, FF), lambda i: (i, 0)),
        out_shape=jax.ShapeDtypeStruct((NN, FF), jnp.float32),
    )(acc, hlin, dinv, b)


def kernel(x, edge_index, W1, b1, W2, b2, W3, b3):
    src = edge_index[0]
    dst = edge_index[1]
    pad = EPAD - EE
    srcp = jnp.concatenate(
        [src, jnp.zeros((pad,), jnp.int32)]).reshape(NWORK, CCH, KCH)
    dstp = jnp.concatenate(
        [dst, jnp.full((pad,), NN, jnp.int32)]).reshape(NWORK, CCH, KCH)

    zeros_rows = jnp.zeros((RPT, FF), jnp.float32)
    ones_rows = jnp.ones((KCH, FF), jnp.float32)

    deg_parts = _sc_deg(dstp, zeros_rows, ones_rows)
    dinv = _tc_prep(deg_parts)

    hlin1, g1 = _tc_mm1(x, W1, dinv)
    acc1 = _sc_spmm(g1, srcp, dstp, zeros_rows)
    hlin2, g2 = _tc_mid(acc1, hlin1, dinv, b1.reshape(1, FF), W2)
    acc2 = _sc_spmm(g2, srcp, dstp, zeros_rows)
    hlin3, g3 = _tc_mid(acc2, hlin2, dinv, b2.reshape(1, FF), W3)
    acc3 = _sc_spmm(g3, srcp, dstp, zeros_rows)
    return _tc_final(acc3, hlin3, dinv, b3.reshape(1, FF))


# speedup vs baseline: 6.9106x; 1.0808x over previous
import functools
import jax
import jax.numpy as jnp
from jax import lax
from jax.experimental import pallas as pl
from jax.experimental.pallas import tpu as pltpu
from jax.experimental.pallas import tpu_sc as plsc

NN = 10000
FF = 128
EE = 320000

NCORE = 2
NSUB = 16
NWORK = NCORE * NSUB
KCH = 128
CCH = 80
EPAD = NWORK * CCH * KCH
RPT = 632
ACC_ROWS = NSUB * RPT

BLK = 2000

_mesh = plsc.VectorSubcoreMesh(core_axis_name="c", subcore_axis_name="s")


def _zero_acc(zeros_hbm, acc_sh, s):
    for r in range(RPT // KCH):
        pltpu.sync_copy(zeros_hbm, acc_sh.at[pl.ds(s * RPT + r * KCH, KCH)])
    rem = RPT % KCH
    pltpu.sync_copy(zeros_hbm.at[pl.ds(0, rem)],
                    acc_sh.at[pl.ds(s * RPT + (RPT - rem), rem)])


def _unpack_chunk(pidx_v, q, sidx2_v, didx2_v, b):
    for kk in range(KCH // 16):
        v = pidx_v[q, pl.ds(kk * 16, 16)]
        sidx2_v[b, pl.ds(kk * 16, 16)] = v & jnp.int32(0xFFFF)
        didx2_v[b, pl.ds(kk * 16, 16)] = lax.shift_right_logical(v, 16)


@functools.partial(
    pl.kernel,
    mesh=_mesh,
    out_type=jax.ShapeDtypeStruct((NCORE, ACC_ROWS, FF), jnp.float32),
    scratch_types=[
        pltpu.VMEM((CCH, KCH), jnp.int32),
        pltpu.VMEM((KCH, FF), jnp.float32),
        pltpu.VMEM_SHARED((ACC_ROWS, FF), jnp.float32),
    ],
)
def _sc_deg(pk_hbm, zeros_hbm, ones_hbm, out_hbm, pidx_v, ones_v, deg_sh):
    c = lax.axis_index("c")
    s = lax.axis_index("s")
    w = s * NCORE + c
    _zero_acc(zeros_hbm, deg_sh, s)
    pltpu.sync_copy(pk_hbm.at[w], pidx_v)
    pltpu.sync_copy(ones_hbm, ones_v)

    def unp(r, carry):
        for kk in range(KCH // 16):
            v = pidx_v[r, pl.ds(kk * 16, 16)]
            pidx_v[r, pl.ds(kk * 16, 16)] = lax.shift_right_logical(v, 16)
        return carry

    lax.fori_loop(0, CCH, unp, 0)
    plsc.subcore_barrier()

    def body(i, carry):
        pltpu.sync_copy(ones_v, deg_sh.at[pidx_v.at[i]], add=True)
        return carry

    lax.fori_loop(0, CCH, body, 0)
    plsc.subcore_barrier()
    pltpu.sync_copy(deg_sh.at[pl.ds(s * RPT, RPT)],
                    out_hbm.at[c, pl.ds(s * RPT, RPT)])


@functools.partial(
    pl.kernel,
    mesh=_mesh,
    out_type=jax.ShapeDtypeStruct((NCORE, ACC_ROWS, FF), jnp.float32),
    scratch_types=[
        pltpu.VMEM((CCH, KCH), jnp.int32),
        pltpu.VMEM((2, KCH), jnp.int32),
        pltpu.VMEM((2, KCH), jnp.int32),
        pltpu.VMEM((KCH, FF), jnp.float32),
        pltpu.VMEM((KCH, FF), jnp.float32),
        pltpu.VMEM_SHARED((ACC_ROWS, FF), jnp.float32),
        pltpu.SemaphoreType.DMA,
        pltpu.SemaphoreType.DMA,
    ],
)
def _sc_spmm(g_hbm, pk_hbm, zeros_hbm, out_hbm,
             pidx_v, sidx2_v, didx2_v, rows0_v, rows1_v, acc_sh, sem0, sem1):
    c = lax.axis_index("c")
    s = lax.axis_index("s")
    w = s * NCORE + c
    _zero_acc(zeros_hbm, acc_sh, s)
    pltpu.sync_copy(pk_hbm.at[w], pidx_v)
    plsc.subcore_barrier()

    bufs = (rows0_v, rows1_v)
    sems = (sem0, sem1)
    _unpack_chunk(pidx_v, 0, sidx2_v, didx2_v, 0)
    pltpu.async_copy(g_hbm.at[sidx2_v.at[0]], bufs[0], sems[0])

    def body(i, carry):
        for b in range(2):
            q = i * 2 + b
            nb = (b + 1) % 2

            @pl.when(q + 1 < CCH)
            def _():
                _unpack_chunk(pidx_v, q + 1, sidx2_v, didx2_v, nb)
                pltpu.async_copy(g_hbm.at[sidx2_v.at[nb]], bufs[nb], sems[nb])

            pltpu.make_async_copy(g_hbm.at[pl.ds(0, KCH)], bufs[b],
                                  sems[b]).wait()
            pltpu.sync_copy(bufs[b], acc_sh.at[didx2_v.at[b]], add=True)
        return carry

    lax.fori_loop(0, CCH // 2, body, 0)
    plsc.subcore_barrier()
    pltpu.sync_copy(acc_sh.at[pl.ds(s * RPT, RPT)],
                    out_hbm.at[c, pl.ds(s * RPT, RPT)])


def _prep_body(parts_ref, dinv_ref):
    deg = parts_ref[0, :NN, :1] + parts_ref[1, :NN, :1] + 1.0
    dinv_ref[...] = lax.rsqrt(deg)


def _tc_prep(deg_parts):
    return pl.pallas_call(
        _prep_body,
        out_shape=jax.ShapeDtypeStruct((NN, 1), jnp.float32),
    )(deg_parts)


def _mm1_body(x_ref, w_ref, dinv_ref, hlin_ref, g_ref):
    hl = jnp.dot(x_ref[...], w_ref[...],
                 preferred_element_type=jnp.float32,
                 precision=lax.Precision.HIGHEST)
    hlin_ref[...] = hl
    g_ref[...] = hl * dinv_ref[...]


def _tc_mm1(x, w, dinv):
    return pl.pallas_call(
        _mm1_body,
        grid=(NN // BLK,),
        in_specs=[
            pl.BlockSpec((BLK, FF), lambda i: (i, 0)),
            pl.BlockSpec((FF, FF), lambda i: (0, 0)),
            pl.BlockSpec((BLK, 1), lambda i: (i, 0)),
        ],
        out_specs=[
            pl.BlockSpec((BLK, FF), lambda i: (i, 0)),
            pl.BlockSpec((BLK, FF), lambda i: (i, 0)),
        ],
        out_shape=[
            jax.ShapeDtypeStruct((NN, FF), jnp.float32),
            jax.ShapeDtypeStruct((NN, FF), jnp.float32),
        ],
    )(x, w, dinv)


def _mid_body(acc_ref, hlin_ref, dinv_ref, b_ref, w_ref, hlinn_ref, gn_ref):
    dv = dinv_ref[...]
    a = acc_ref[0] + acc_ref[1]
    sfull = dv * a + (dv * dv) * hlin_ref[...] + b_ref[...]
    h = jnp.maximum(sfull, 0.0)
    hn = jnp.dot(h, w_ref[...],
                 preferred_element_type=jnp.float32,
                 precision=lax.Precision.HIGHEST)
    hlinn_ref[...] = hn
    gn_ref[...] = hn * dv


def _tc_mid(acc, hlin, dinv, b, w):
    return pl.pallas_call(
        _mid_body,
        grid=(NN // BLK,),
        in_specs=[
            pl.BlockSpec((NCORE, BLK, FF), lambda i: (0, i, 0)),
            pl.BlockSpec((BLK, FF), lambda i: (i, 0)),
            pl.BlockSpec((BLK, 1), lambda i: (i, 0)),
            pl.BlockSpec((1, FF), lambda i: (0, 0)),
            pl.BlockSpec((FF, FF), lambda i: (0, 0)),
        ],
        out_specs=[
            pl.BlockSpec((BLK, FF), lambda i: (i, 0)),
            pl.BlockSpec((BLK, FF), lambda i: (i, 0)),
        ],
        out_shape=[
            jax.ShapeDtypeStruct((NN, FF), jnp.float32),
            jax.ShapeDtypeStruct((NN, FF), jnp.float32),
        ],
    )(acc, hlin, dinv, b, w)


def _final_body(acc_ref, hlin_ref, dinv_ref, b_ref, out_ref):
    dv = dinv_ref[...]
    a = acc_ref[0] + acc_ref[1]
    out_ref[...] = dv * a + (dv * dv) * hlin_ref[...] + b_ref[...]


def _tc_final(acc, hlin, dinv, b):
    return pl.pallas_call(
        _final_body,
        grid=(NN // BLK,),
        in_specs=[
            pl.BlockSpec((NCORE, BLK, FF), lambda i: (0, i, 0)),
            pl.BlockSpec((BLK, FF), lambda i: (i, 0)),
            pl.BlockSpec((BLK, 1), lambda i: (i, 0)),
            pl.BlockSpec((1, FF), lambda i: (0, 0)),
        ],
        out_specs=pl.BlockSpec((BLK, FF), lambda i: (i, 0)),
        out_shape=jax.ShapeDtypeStruct((NN, FF), jnp.float32),
    )(acc, hlin, dinv, b)


def kernel(x, edge_index, W1, b1, W2, b2, W3, b3):
    src = edge_index[0]
    dst = edge_index[1]
    pad = EPAD - EE
    packed = jnp.left_shift(dst, 16) | src
    pk = jnp.concatenate(
        [packed, jnp.full((pad,), NN << 16, jnp.int32)]).reshape(
            NWORK, CCH, KCH)

    zeros_rows = jnp.zeros((KCH, FF), jnp.float32)
    ones_rows = jnp.ones((KCH, FF), jnp.float32)

    deg_parts = _sc_deg(pk, zeros_rows, ones_rows)
    dinv = _tc_prep(deg_parts)

    hlin1, g1 = _tc_mm1(x, W1, dinv)
    acc1 = _sc_spmm(g1, pk, zeros_rows)
    hlin2, g2 = _tc_mid(acc1, hlin1, dinv, b1.reshape(1, FF), W2)
    acc2 = _sc_spmm(g2, pk, zeros_rows)
    hlin3, g3 = _tc_mid(acc2, hlin2, dinv, b2.reshape(1, FF), W3)
    acc3 = _sc_spmm(g3, pk, zeros_rows)
    return _tc_final(acc3, hlin3, dinv, b3.reshape(1, FF))


# speedup vs baseline: 7.5571x; 1.0935x over previous
import functools
import jax
import jax.numpy as jnp
from jax import lax
from jax.experimental import pallas as pl
from jax.experimental.pallas import tpu as pltpu
from jax.experimental.pallas import tpu_sc as plsc

NN = 10000
FF = 128
EE = 320000

NCORE = 2
NSUB = 16
NWORK = NCORE * NSUB
KCH = 128
CCH = 80
EPAD = NWORK * CCH * KCH
NCHUNK = EPAD // KCH
CA = 136
CB = 2 * CCH - CA
RPT = 632
ACC_ROWS = NN + 8
RLAST = ACC_ROWS - (NSUB - 1) * RPT

BLK = 2000

_mesh = plsc.VectorSubcoreMesh(core_axis_name="c", subcore_axis_name="s")


def _zero_acc(zeros_hbm, acc_sh, s):
    def fill(nrows):
        for r in range(nrows // KCH):
            pltpu.sync_copy(zeros_hbm,
                            acc_sh.at[pl.ds(s * RPT + r * KCH, KCH)])
        rem = nrows % KCH
        pltpu.sync_copy(zeros_hbm.at[pl.ds(0, rem)],
                        acc_sh.at[pl.ds(s * RPT + (nrows - rem), rem)])

    @pl.when(s < NSUB - 1)
    def _():
        fill(RPT)

    @pl.when(s == NSUB - 1)
    def _():
        fill(RLAST)


def _copy_out(acc_sh, out_hbm, c, s):
    @pl.when(s < NSUB - 1)
    def _():
        pltpu.sync_copy(acc_sh.at[pl.ds(s * RPT, RPT)],
                        out_hbm.at[c, pl.ds(s * RPT, RPT)])

    @pl.when(s == NSUB - 1)
    def _():
        pltpu.sync_copy(acc_sh.at[pl.ds((NSUB - 1) * RPT, RLAST)],
                        out_hbm.at[c, pl.ds((NSUB - 1) * RPT, RLAST)])


def _unpack_chunk(pidx_v, q, sidx2_v, didx2_v, b):
    for kk in range(KCH // 16):
        v = pidx_v[q, pl.ds(kk * 16, 16)]
        sidx2_v[b, pl.ds(kk * 16, 16)] = v & jnp.int32(0xFFFF)
        didx2_v[b, pl.ds(kk * 16, 16)] = lax.shift_right_logical(v, 16)


@functools.partial(
    pl.kernel,
    mesh=_mesh,
    out_type=jax.ShapeDtypeStruct((NCORE, ACC_ROWS, FF), jnp.float32),
    scratch_types=[
        pltpu.VMEM((CCH, KCH), jnp.int32),
        pltpu.VMEM((KCH, FF), jnp.float32),
        pltpu.VMEM_SHARED((ACC_ROWS, FF), jnp.float32),
    ],
)
def _sc_deg(pk_hbm, zeros_hbm, ones_hbm, out_hbm, pidx_v, ones_v, deg_sh):
    c = lax.axis_index("c")
    s = lax.axis_index("s")
    w = s * NCORE + c
    _zero_acc(zeros_hbm, deg_sh, s)
    pltpu.sync_copy(pk_hbm.at[pl.ds(w * CCH, CCH)], pidx_v)
    pltpu.sync_copy(ones_hbm, ones_v)

    def unp(r, carry):
        for kk in range(KCH // 16):
            v = pidx_v[r, pl.ds(kk * 16, 16)]
            pidx_v[r, pl.ds(kk * 16, 16)] = lax.shift_right_logical(v, 16)
        return carry

    lax.fori_loop(0, CCH, unp, 0)
    plsc.subcore_barrier()

    def body(i, carry):
        pltpu.sync_copy(ones_v, deg_sh.at[pidx_v.at[i]], add=True)
        return carry

    lax.fori_loop(0, CCH, body, 0)
    plsc.subcore_barrier()
    _copy_out(deg_sh, out_hbm, c, s)


@functools.partial(
    pl.kernel,
    mesh=_mesh,
    out_type=jax.ShapeDtypeStruct((NCORE, ACC_ROWS, FF), jnp.float32),
    scratch_types=[
        pltpu.VMEM((CA, KCH), jnp.int32),
        pltpu.VMEM((2, KCH), jnp.int32),
        pltpu.VMEM((2, KCH), jnp.int32),
        pltpu.VMEM((KCH, FF), jnp.float32),
        pltpu.VMEM((KCH, FF), jnp.float32),
        pltpu.VMEM_SHARED((ACC_ROWS, FF), jnp.float32),
        pltpu.SemaphoreType.DMA,
        pltpu.SemaphoreType.DMA,
    ],
)
def _sc_spmm(g_hbm, pk_hbm, zeros_hbm, out_hbm,
             pidx_v, sidx2_v, didx2_v, rows0_v, rows1_v, acc_sh, sem0, sem1):
    c = lax.axis_index("c")
    s = lax.axis_index("s")
    bufs = (rows0_v, rows1_v)
    sems = (sem0, sem1)
    _zero_acc(zeros_hbm, acc_sh, s)

    def pre(nch, off):
        pltpu.sync_copy(pk_hbm.at[pl.ds(off, nch)], pidx_v.at[pl.ds(0, nch)])
        _unpack_chunk(pidx_v, 0, sidx2_v, didx2_v, 0)
        pltpu.async_copy(g_hbm.at[sidx2_v.at[0]], bufs[0], sems[0])

    def run(nch):
        def body(i, carry):
            for b in range(2):
                q = i * 2 + b
                nb = (b + 1) % 2

                @pl.when(q + 1 < nch)
                def _():
                    _unpack_chunk(pidx_v, q + 1, sidx2_v, didx2_v, nb)
                    pltpu.async_copy(g_hbm.at[sidx2_v.at[nb]], bufs[nb],
                                     sems[nb])

                pltpu.make_async_copy(g_hbm.at[pl.ds(0, KCH)], bufs[b],
                                      sems[b]).wait()
                pltpu.sync_copy(bufs[b], acc_sh.at[didx2_v.at[b]], add=True)
            return carry

        lax.fori_loop(0, nch // 2, body, 0)

    @pl.when(c == 0)
    def _():
        pre(CA, s * CA)

    @pl.when(c == 1)
    def _():
        pre(CB, NSUB * CA + s * CB)

    plsc.subcore_barrier()

    @pl.when(c == 0)
    def _():
        run(CA)

    @pl.when(c == 1)
    def _():
        run(CB)

    plsc.subcore_barrier()
    _copy_out(acc_sh, out_hbm, c, s)


def _prep_body(parts_ref, dinv_ref):
    deg = parts_ref[0, :NN, :1] + parts_ref[1, :NN, :1] + 1.0
    dinv_ref[...] = lax.rsqrt(deg)


def _tc_prep(deg_parts):
    return pl.pallas_call(
        _prep_body,
        out_shape=jax.ShapeDtypeStruct((NN, 1), jnp.float32),
    )(deg_parts)


def _mm1_body(x_ref, w_ref, dinv_ref, hlin_ref, g_ref):
    hl = jnp.dot(x_ref[...], w_ref[...],
                 preferred_element_type=jnp.float32,
                 precision=lax.Precision.HIGHEST)
    hlin_ref[...] = hl
    g_ref[...] = hl * dinv_ref[...]


def _tc_mm1(x, w, dinv):
    return pl.pallas_call(
        _mm1_body,
        grid=(NN // BLK,),
        in_specs=[
            pl.BlockSpec((BLK, FF), lambda i: (i, 0)),
            pl.BlockSpec((FF, FF), lambda i: (0, 0)),
            pl.BlockSpec((BLK, 1), lambda i: (i, 0)),
        ],
        out_specs=[
            pl.BlockSpec((BLK, FF), lambda i: (i, 0)),
            pl.BlockSpec((BLK, FF), lambda i: (i, 0)),
        ],
        out_shape=[
            jax.ShapeDtypeStruct((NN, FF), jnp.float32),
            jax.ShapeDtypeStruct((NN, FF), jnp.float32),
        ],
    )(x, w, dinv)


def _mid_body(acc_ref, hlin_ref, dinv_ref, b_ref, w_ref, hlinn_ref, gn_ref):
    dv = dinv_ref[...]
    a = acc_ref[0] + acc_ref[1]
    sfull = dv * a + (dv * dv) * hlin_ref[...] + b_ref[...]
    h = jnp.maximum(sfull, 0.0)
    hn = jnp.dot(h, w_ref[...],
                 preferred_element_type=jnp.float32,
                 precision=lax.Precision.HIGHEST)
    hlinn_ref[...] = hn
    gn_ref[...] = hn * dv


def _tc_mid(acc, hlin, dinv, b, w):
    return pl.pallas_call(
        _mid_body,
        grid=(NN // BLK,),
        in_specs=[
            pl.BlockSpec((NCORE, BLK, FF), lambda i: (0, i, 0)),
            pl.BlockSpec((BLK, FF), lambda i: (i, 0)),
            pl.BlockSpec((BLK, 1), lambda i: (i, 0)),
            pl.BlockSpec((1, FF), lambda i: (0, 0)),
            pl.BlockSpec((FF, FF), lambda i: (0, 0)),
        ],
        out_specs=[
            pl.BlockSpec((BLK, FF), lambda i: (i, 0)),
            pl.BlockSpec((BLK, FF), lambda i: (i, 0)),
        ],
        out_shape=[
            jax.ShapeDtypeStruct((NN, FF), jnp.float32),
            jax.ShapeDtypeStruct((NN, FF), jnp.float32),
        ],
    )(acc, hlin, dinv, b, w)


def _final_body(acc_ref, hlin_ref, dinv_ref, b_ref, out_ref):
    dv = dinv_ref[...]
    a = acc_ref[0] + acc_ref[1]
    out_ref[...] = dv * a + (dv * dv) * hlin_ref[...] + b_ref[...]


def _tc_final(acc, hlin, dinv, b):
    return pl.pallas_call(
        _final_body,
        grid=(NN // BLK,),
        in_specs=[
            pl.BlockSpec((NCORE, BLK, FF), lambda i: (0, i, 0)),
            pl.BlockSpec((BLK, FF), lambda i: (i, 0)),
            pl.BlockSpec((BLK, 1), lambda i: (i, 0)),
            pl.BlockSpec((1, FF), lambda i: (0, 0)),
        ],
        out_specs=pl.BlockSpec((BLK, FF), lambda i: (i, 0)),
        out_shape=jax.ShapeDtypeStruct((NN, FF), jnp.float32),
    )(acc, hlin, dinv, b)


def kernel(x, edge_index, W1, b1, W2, b2, W3, b3):
    src = edge_index[0]
    dst = edge_index[1]
    pad = EPAD - EE
    packed = jnp.left_shift(dst, 16) | src
    pk = jnp.concatenate(
        [packed, jnp.full((pad,), NN << 16, jnp.int32)]).reshape(NCHUNK, KCH)

    zeros_rows = jnp.zeros((KCH, FF), jnp.float32)
    ones_rows = jnp.ones((KCH, FF), jnp.float32)

    deg_parts = _sc_deg(pk, zeros_rows, ones_rows)
    dinv = _tc_prep(deg_parts)

    hlin1, g1 = _tc_mm1(x, W1, dinv)
    acc1 = _sc_spmm(g1, pk, zeros_rows)
    hlin2, g2 = _tc_mid(acc1, hlin1, dinv, b1.reshape(1, FF), W2)
    acc2 = _sc_spmm(g2, pk, zeros_rows)
    hlin3, g3 = _tc_mid(acc2, hlin2, dinv, b2.reshape(1, FF), W3)
    acc3 = _sc_spmm(g3, pk, zeros_rows)
    return _tc_final(acc3, hlin3, dinv, b3.reshape(1, FF))


# speedup vs baseline: 7.6417x; 1.0112x over previous
import functools
import jax
import jax.numpy as jnp
from jax import lax
from jax.experimental import pallas as pl
from jax.experimental.pallas import tpu as pltpu
from jax.experimental.pallas import tpu_sc as plsc

NN = 10000
FF = 128
EE = 320000

NCORE = 2
NSUB = 16
NWORK = NCORE * NSUB
KCH = 128
CCH = 80
EPAD = NWORK * CCH * KCH
NCHUNK = EPAD // KCH
CA = 136
CB = 2 * CCH - CA
RPT = 632
ACC_ROWS = NN + 8
RLAST = ACC_ROWS - (NSUB - 1) * RPT

BLK = 2000

_mesh = plsc.VectorSubcoreMesh(core_axis_name="c", subcore_axis_name="s")


def _zero_acc(zeros_hbm, acc_sh, s):
    def fill(nrows):
        for r in range(nrows // KCH):
            pltpu.sync_copy(zeros_hbm,
                            acc_sh.at[pl.ds(s * RPT + r * KCH, KCH)])
        rem = nrows % KCH
        pltpu.sync_copy(zeros_hbm.at[pl.ds(0, rem)],
                        acc_sh.at[pl.ds(s * RPT + (nrows - rem), rem)])

    @pl.when(s < NSUB - 1)
    def _():
        fill(RPT)

    @pl.when(s == NSUB - 1)
    def _():
        fill(RLAST)


def _copy_out(acc_sh, out_hbm, c, s):
    @pl.when(s < NSUB - 1)
    def _():
        pltpu.sync_copy(acc_sh.at[pl.ds(s * RPT, RPT)],
                        out_hbm.at[c, pl.ds(s * RPT, RPT)])

    @pl.when(s == NSUB - 1)
    def _():
        pltpu.sync_copy(acc_sh.at[pl.ds((NSUB - 1) * RPT, RLAST)],
                        out_hbm.at[c, pl.ds((NSUB - 1) * RPT, RLAST)])


def _unpack_chunk(pidx_v, q, sidx2_v, didx2_v, b):
    for kk in range(KCH // 16):
        v = pidx_v[q, pl.ds(kk * 16, 16)]
        sidx2_v[b, pl.ds(kk * 16, 16)] = v & jnp.int32(0xFFFF)
        didx2_v[b, pl.ds(kk * 16, 16)] = lax.shift_right_logical(v, 16)


@functools.partial(
    pl.kernel,
    mesh=_mesh,
    out_type=jax.ShapeDtypeStruct((NCORE, ACC_ROWS, FF), jnp.float32),
    scratch_types=[
        pltpu.VMEM((CCH, KCH), jnp.int32),
        pltpu.VMEM((KCH, FF), jnp.float32),
        pltpu.VMEM_SHARED((ACC_ROWS, FF), jnp.float32),
    ],
)
def _sc_deg(pk_hbm, zeros_hbm, ones_hbm, out_hbm, pidx_v, ones_v, deg_sh):
    c = lax.axis_index("c")
    s = lax.axis_index("s")
    w = s * NCORE + c
    _zero_acc(zeros_hbm, deg_sh, s)
    pltpu.sync_copy(pk_hbm.at[pl.ds(w * CCH, CCH)], pidx_v)
    pltpu.sync_copy(ones_hbm, ones_v)

    def unp(r, carry):
        for kk in range(KCH // 16):
            v = pidx_v[r, pl.ds(kk * 16, 16)]
            pidx_v[r, pl.ds(kk * 16, 16)] = lax.shift_right_logical(v, 16)
        return carry

    lax.fori_loop(0, CCH, unp, 0)
    plsc.subcore_barrier()

    def body(i, carry):
        pltpu.sync_copy(ones_v, deg_sh.at[pidx_v.at[i]], add=True)
        return carry

    lax.fori_loop(0, CCH, body, 0)
    plsc.subcore_barrier()
    _copy_out(deg_sh, out_hbm, c, s)


@functools.partial(
    pl.kernel,
    mesh=_mesh,
    out_type=jax.ShapeDtypeStruct((NCORE, ACC_ROWS, FF), jnp.float32),
    scratch_types=[
        pltpu.VMEM((CA, KCH), jnp.int32),
        pltpu.VMEM((2, KCH), jnp.int32),
        pltpu.VMEM((2, KCH), jnp.int32),
        pltpu.VMEM((KCH, FF), jnp.float32),
        pltpu.VMEM((KCH, FF), jnp.float32),
        pltpu.VMEM_SHARED((ACC_ROWS, FF), jnp.float32),
        pltpu.SemaphoreType.DMA,
        pltpu.SemaphoreType.DMA,
    ],
)
def _sc_spmm(g_hbm, pk_hbm, zeros_hbm, out_hbm,
             pidx_v, sidx2_v, didx2_v, rows0_v, rows1_v, acc_sh, sem0, sem1):
    c = lax.axis_index("c")
    s = lax.axis_index("s")
    bufs = (rows0_v, rows1_v)
    sems = (sem0, sem1)
    with jax.named_scope("ph_zero"):
        _zero_acc(zeros_hbm, acc_sh, s)

    def pre(nch, off):
        pltpu.sync_copy(pk_hbm.at[pl.ds(off, nch)], pidx_v.at[pl.ds(0, nch)])
        _unpack_chunk(pidx_v, 0, sidx2_v, didx2_v, 0)
        pltpu.async_copy(g_hbm.at[sidx2_v.at[0]], bufs[0], sems[0])

    def run(nch):
        def body(i, carry):
            for b in range(2):
                q = i * 2 + b
                nb = (b + 1) % 2

                @pl.when(q + 1 < nch)
                def _():
                    _unpack_chunk(pidx_v, q + 1, sidx2_v, didx2_v, nb)
                    pltpu.async_copy(g_hbm.at[sidx2_v.at[nb]], bufs[nb],
                                     sems[nb])

                pltpu.make_async_copy(g_hbm.at[pl.ds(0, KCH)], bufs[b],
                                      sems[b]).wait()
                pltpu.sync_copy(bufs[b], acc_sh.at[didx2_v.at[b]], add=True)
            return carry

        lax.fori_loop(0, nch // 2, body, 0)

    with jax.named_scope("ph_pre"):
        @pl.when(c == 0)
        def _():
            pre(CA, s * CA)

        @pl.when(c == 1)
        def _():
            pre(CB, NSUB * CA + s * CB)

        plsc.subcore_barrier()

    with jax.named_scope("ph_loop"):
        @pl.when(c == 0)
        def _():
            run(CA)

        @pl.when(c == 1)
        def _():
            run(CB)

        plsc.subcore_barrier()

    with jax.named_scope("ph_out"):
        _copy_out(acc_sh, out_hbm, c, s)


def _prep_body(parts_ref, dinv_ref):
    deg = parts_ref[0, :NN, :1] + parts_ref[1, :NN, :1] + 1.0
    dinv_ref[...] = lax.rsqrt(deg)


def _tc_prep(deg_parts):
    return pl.pallas_call(
        _prep_body,
        out_shape=jax.ShapeDtypeStruct((NN, 1), jnp.float32),
    )(deg_parts)


def _mm1_body(x_ref, w_ref, dinv_ref, hlin_ref, g_ref):
    hl = jnp.dot(x_ref[...], w_ref[...],
                 preferred_element_type=jnp.float32,
                 precision=lax.Precision.HIGHEST)
    hlin_ref[...] = hl
    g_ref[...] = hl * dinv_ref[...]


def _tc_mm1(x, w, dinv):
    return pl.pallas_call(
        _mm1_body,
        grid=(NN // BLK,),
        in_specs=[
            pl.BlockSpec((BLK, FF), lambda i: (i, 0)),
            pl.BlockSpec((FF, FF), lambda i: (0, 0)),
            pl.BlockSpec((BLK, 1), lambda i: (i, 0)),
        ],
        out_specs=[
            pl.BlockSpec((BLK, FF), lambda i: (i, 0)),
            pl.BlockSpec((BLK, FF), lambda i: (i, 0)),
        ],
        out_shape=[
            jax.ShapeDtypeStruct((NN, FF), jnp.float32),
            jax.ShapeDtypeStruct((NN, FF), jnp.float32),
        ],
    )(x, w, dinv)


def _mid_body(acc_ref, hlin_ref, dinv_ref, b_ref, w_ref, hlinn_ref, gn_ref):
    dv = dinv_ref[...]
    a = acc_ref[0] + acc_ref[1]
    sfull = dv * a + (dv * dv) * hlin_ref[...] + b_ref[...]
    h = jnp.maximum(sfull, 0.0)
    hn = jnp.dot(h, w_ref[...],
                 preferred_element_type=jnp.float32,
                 precision=lax.Precision.HIGHEST)
    hlinn_ref[...] = hn
    gn_ref[...] = hn * dv


def _tc_mid(acc, hlin, dinv, b, w):
    return pl.pallas_call(
        _mid_body,
        grid=(NN // BLK,),
        in_specs=[
            pl.BlockSpec((NCORE, BLK, FF), lambda i: (0, i, 0)),
            pl.BlockSpec((BLK, FF), lambda i: (i, 0)),
            pl.BlockSpec((BLK, 1), lambda i: (i, 0)),
            pl.BlockSpec((1, FF), lambda i: (0, 0)),
            pl.BlockSpec((FF, FF), lambda i: (0, 0)),
        ],
        out_specs=[
            pl.BlockSpec((BLK, FF), lambda i: (i, 0)),
            pl.BlockSpec((BLK, FF), lambda i: (i, 0)),
        ],
        out_shape=[
            jax.ShapeDtypeStruct((NN, FF), jnp.float32),
            jax.ShapeDtypeStruct((NN, FF), jnp.float32),
        ],
    )(acc, hlin, dinv, b, w)


def _final_body(acc_ref, hlin_ref, dinv_ref, b_ref, out_ref):
    dv = dinv_ref[...]
    a = acc_ref[0] + acc_ref[1]
    out_ref[...] = dv * a + (dv * dv) * hlin_ref[...] + b_ref[...]


def _tc_final(acc, hlin, dinv, b):
    return pl.pallas_call(
        _final_body,
        grid=(NN // BLK,),
        in_specs=[
            pl.BlockSpec((NCORE, BLK, FF), lambda i: (0, i, 0)),
            pl.BlockSpec((BLK, FF), lambda i: (i, 0)),
            pl.BlockSpec((BLK, 1), lambda i: (i, 0)),
            pl.BlockSpec((1, FF), lambda i: (0, 0)),
        ],
        out_specs=pl.BlockSpec((BLK, FF), lambda i: (i, 0)),
        out_shape=jax.ShapeDtypeStruct((NN, FF), jnp.float32),
    )(acc, hlin, dinv, b)


def kernel(x, edge_index, W1, b1, W2, b2, W3, b3):
    src = edge_index[0]
    dst = edge_index[1]
    pad = EPAD - EE
    packed = jnp.left_shift(dst, 16) | src
    pk = jnp.concatenate(
        [packed, jnp.full((pad,), NN << 16, jnp.int32)]).reshape(NCHUNK, KCH)

    zeros_rows = jnp.zeros((KCH, FF), jnp.float32)
    ones_rows = jnp.ones((KCH, FF), jnp.float32)

    deg_parts = _sc_deg(pk, zeros_rows, ones_rows)
    dinv = _tc_prep(deg_parts)

    hlin1, g1 = _tc_mm1(x, W1, dinv)
    acc1 = _sc_spmm(g1, pk, zeros_rows)
    hlin2, g2 = _tc_mid(acc1, hlin1, dinv, b1.reshape(1, FF), W2)
    acc2 = _sc_spmm(g2, pk, zeros_rows)
    hlin3, g3 = _tc_mid(acc2, hlin2, dinv, b2.reshape(1, FF), W3)
    acc3 = _sc_spmm(g3, pk, zeros_rows)
    return _tc_final(acc3, hlin3, dinv, b3.reshape(1, FF))


# speedup vs baseline: 9.3490x; 1.2234x over previous
import functools
import jax
import jax.numpy as jnp
from jax import lax
from jax.experimental import pallas as pl
from jax.experimental.pallas import tpu as pltpu
from jax.experimental.pallas import tpu_sc as plsc

NN = 10000
FF = 128
EE = 320000

NCORE = 2
NSUB = 16
NWORK = NCORE * NSUB
KCH = 128
CCH = 80
EPAD = NWORK * CCH * KCH
NCHUNK = EPAD // KCH
CA = 152
CB = 2 * CCH - CA
RPT = 632
ACC_ROWS = NN + 8
RLAST = ACC_ROWS - (NSUB - 1) * RPT

BLK = 2000

_mesh = plsc.VectorSubcoreMesh(core_axis_name="c", subcore_axis_name="s")


def _zero_acc(zeros_hbm, acc_sh, s):
    def fill(nrows):
        for r in range(nrows // KCH):
            pltpu.sync_copy(zeros_hbm,
                            acc_sh.at[pl.ds(s * RPT + r * KCH, KCH)])
        rem = nrows % KCH
        pltpu.sync_copy(zeros_hbm.at[pl.ds(0, rem)],
                        acc_sh.at[pl.ds(s * RPT + (nrows - rem), rem)])

    @pl.when(s < NSUB - 1)
    def _():
        fill(RPT)

    @pl.when(s == NSUB - 1)
    def _():
        fill(RLAST)


def _copy_out(acc_sh, out_hbm, c, s):
    @pl.when(s < NSUB - 1)
    def _():
        pltpu.sync_copy(acc_sh.at[pl.ds(s * RPT, RPT)],
                        out_hbm.at[c, pl.ds(s * RPT, RPT)])

    @pl.when(s == NSUB - 1)
    def _():
        pltpu.sync_copy(acc_sh.at[pl.ds((NSUB - 1) * RPT, RLAST)],
                        out_hbm.at[c, pl.ds((NSUB - 1) * RPT, RLAST)])


def _unpack_chunk(pk2_v, sidx2_v, didx2_v, b):
    for kk in range(KCH // 16):
        v = pk2_v[b, 0, pl.ds(kk * 16, 16)]
        sidx2_v[b, pl.ds(kk * 16, 16)] = v & jnp.int32(0xFFFF)
        didx2_v[b, pl.ds(kk * 16, 16)] = lax.shift_right_logical(v, 16)


@functools.partial(
    pl.kernel,
    mesh=_mesh,
    out_type=jax.ShapeDtypeStruct((NCORE, ACC_ROWS, FF), jnp.float32),
    scratch_types=[
        pltpu.VMEM((CCH, KCH), jnp.int32),
        pltpu.VMEM((KCH, FF), jnp.float32),
        pltpu.VMEM_SHARED((ACC_ROWS, FF), jnp.float32),
    ],
)
def _sc_deg(pk_hbm, zeros_hbm, ones_hbm, out_hbm, pidx_v, ones_v, deg_sh):
    c = lax.axis_index("c")
    s = lax.axis_index("s")
    w = s * NCORE + c
    _zero_acc(zeros_hbm, deg_sh, s)
    pltpu.sync_copy(pk_hbm.at[pl.ds(w * CCH, CCH)], pidx_v)
    pltpu.sync_copy(ones_hbm, ones_v)

    def unp(r, carry):
        for kk in range(KCH // 16):
            v = pidx_v[r, pl.ds(kk * 16, 16)]
            pidx_v[r, pl.ds(kk * 16, 16)] = lax.shift_right_logical(v, 16)
        return carry

    lax.fori_loop(0, CCH, unp, 0)
    plsc.subcore_barrier()

    def body(i, carry):
        pltpu.sync_copy(ones_v, deg_sh.at[pidx_v.at[i]], add=True)
        return carry

    lax.fori_loop(0, CCH, body, 0)
    plsc.subcore_barrier()
    _copy_out(deg_sh, out_hbm, c, s)


@functools.partial(
    pl.kernel,
    mesh=_mesh,
    out_type=jax.ShapeDtypeStruct((NCORE, ACC_ROWS, FF), jnp.float32),
    scratch_types=[
        pltpu.VMEM((2, 1, KCH), jnp.int32),
        pltpu.VMEM((2, KCH), jnp.int32),
        pltpu.VMEM((2, KCH), jnp.int32),
        pltpu.VMEM((KCH, FF), jnp.float32),
        pltpu.VMEM((KCH, FF), jnp.float32),
        pltpu.VMEM_SHARED((ACC_ROWS, FF), jnp.float32),
        pltpu.SemaphoreType.DMA,
        pltpu.SemaphoreType.DMA,
    ],
)
def _sc_spmm(g_hbm, pk_hbm, zeros_hbm, out_hbm,
             pk2_v, sidx2_v, didx2_v, rows0_v, rows1_v, acc_sh, sem0, sem1):
    c = lax.axis_index("c")
    s = lax.axis_index("s")
    bufs = (rows0_v, rows1_v)
    sems = (sem0, sem1)
    with jax.named_scope("ph_zero"):
        _zero_acc(zeros_hbm, acc_sh, s)

    def fetch(off, q, b):
        pltpu.sync_copy(pk_hbm.at[pl.ds(off + q, 1)], pk2_v.at[pl.ds(b, 1)])
        _unpack_chunk(pk2_v, sidx2_v, didx2_v, b)

    def pre(off):
        fetch(off, 0, 0)
        pltpu.async_copy(g_hbm.at[sidx2_v.at[0]], bufs[0], sems[0])

    def run(nch, off):
        def body(i, carry):
            for b in range(2):
                q = i * 2 + b
                nb = (b + 1) % 2

                @pl.when(q + 1 < nch)
                def _():
                    fetch(off, q + 1, nb)
                    pltpu.async_copy(g_hbm.at[sidx2_v.at[nb]], bufs[nb],
                                     sems[nb])

                pltpu.make_async_copy(g_hbm.at[pl.ds(0, KCH)], bufs[b],
                                      sems[b]).wait()
                pltpu.sync_copy(bufs[b], acc_sh.at[didx2_v.at[b]], add=True)
            return carry

        lax.fori_loop(0, nch // 2, body, 0)

    with jax.named_scope("ph_pre"):
        @pl.when(c == 0)
        def _():
            pre(s * CA)

        @pl.when(c == 1)
        def _():
            pre(NSUB * CA + s * CB)

        plsc.subcore_barrier()

    with jax.named_scope("ph_loop"):
        @pl.when(c == 0)
        def _():
            run(CA, s * CA)

        @pl.when(c == 1)
        def _():
            run(CB, NSUB * CA + s * CB)

        plsc.subcore_barrier()

    with jax.named_scope("ph_out"):
        _copy_out(acc_sh, out_hbm, c, s)


def _prep_body(parts_ref, dinv_ref):
    deg = parts_ref[0, :NN, :1] + parts_ref[1, :NN, :1] + 1.0
    dinv_ref[...] = lax.rsqrt(deg)


def _tc_prep(deg_parts):
    return pl.pallas_call(
        _prep_body,
        out_shape=jax.ShapeDtypeStruct((NN, 1), jnp.float32),
    )(deg_parts)


def _mm1_body(x_ref, w_ref, dinv_ref, hlin_ref, g_ref):
    hl = jnp.dot(x_ref[...], w_ref[...],
                 preferred_element_type=jnp.float32,
                 precision=lax.Precision.HIGHEST)
    hlin_ref[...] = hl
    g_ref[...] = hl * dinv_ref[...]


def _tc_mm1(x, w, dinv):
    return pl.pallas_call(
        _mm1_body,
        grid=(NN // BLK,),
        in_specs=[
            pl.BlockSpec((BLK, FF), lambda i: (i, 0)),
            pl.BlockSpec((FF, FF), lambda i: (0, 0)),
            pl.BlockSpec((BLK, 1), lambda i: (i, 0)),
        ],
        out_specs=[
            pl.BlockSpec((BLK, FF), lambda i: (i, 0)),
            pl.BlockSpec((BLK, FF), lambda i: (i, 0)),
        ],
        out_shape=[
            jax.ShapeDtypeStruct((NN, FF), jnp.float32),
            jax.ShapeDtypeStruct((NN, FF), jnp.float32),
        ],
    )(x, w, dinv)


def _mid_body(acc_ref, hlin_ref, dinv_ref, b_ref, w_ref, hlinn_ref, gn_ref):
    dv = dinv_ref[...]
    a = acc_ref[0] + acc_ref[1]
    sfull = dv * a + (dv * dv) * hlin_ref[...] + b_ref[...]
    h = jnp.maximum(sfull, 0.0)
    hn = jnp.dot(h, w_ref[...],
                 preferred_element_type=jnp.float32,
                 precision=lax.Precision.HIGHEST)
    hlinn_ref[...] = hn
    gn_ref[...] = hn * dv


def _tc_mid(acc, hlin, dinv, b, w):
    return pl.pallas_call(
        _mid_body,
        grid=(NN // BLK,),
        in_specs=[
            pl.BlockSpec((NCORE, BLK, FF), lambda i: (0, i, 0)),
            pl.BlockSpec((BLK, FF), lambda i: (i, 0)),
            pl.BlockSpec((BLK, 1), lambda i: (i, 0)),
            pl.BlockSpec((1, FF), lambda i: (0, 0)),
            pl.BlockSpec((FF, FF), lambda i: (0, 0)),
        ],
        out_specs=[
            pl.BlockSpec((BLK, FF), lambda i: (i, 0)),
            pl.BlockSpec((BLK, FF), lambda i: (i, 0)),
        ],
        out_shape=[
            jax.ShapeDtypeStruct((NN, FF), jnp.float32),
            jax.ShapeDtypeStruct((NN, FF), jnp.float32),
        ],
    )(acc, hlin, dinv, b, w)


def _final_body(acc_ref, hlin_ref, dinv_ref, b_ref, out_ref):
    dv = dinv_ref[...]
    a = acc_ref[0] + acc_ref[1]
    out_ref[...] = dv * a + (dv * dv) * hlin_ref[...] + b_ref[...]


def _tc_final(acc, hlin, dinv, b):
    return pl.pallas_call(
        _final_body,
        grid=(NN // BLK,),
        in_specs=[
            pl.BlockSpec((NCORE, BLK, FF), lambda i: (0, i, 0)),
            pl.BlockSpec((BLK, FF), lambda i: (i, 0)),
            pl.BlockSpec((BLK, 1), lambda i: (i, 0)),
            pl.BlockSpec((1, FF), lambda i: (0, 0)),
        ],
        out_specs=pl.BlockSpec((BLK, FF), lambda i: (i, 0)),
        out_shape=jax.ShapeDtypeStruct((NN, FF), jnp.float32),
    )(acc, hlin, dinv, b)


def kernel(x, edge_index, W1, b1, W2, b2, W3, b3):
    src = edge_index[0]
    dst = edge_index[1]
    pad = EPAD - EE
    packed = jnp.left_shift(dst, 16) | src
    pkflat = jnp.concatenate(
        [packed, jnp.full((pad,), NN << 16, jnp.int32)])
    pk2d = pkflat.reshape(NCHUNK, KCH)
    pk3d = pkflat.reshape(NCHUNK, 1, KCH)

    zeros_rows = jnp.zeros((KCH, FF), jnp.float32)
    ones_rows = jnp.ones((KCH, FF), jnp.float32)

    deg_parts = _sc_deg(pk2d, zeros_rows, ones_rows)
    dinv = _tc_prep(deg_parts)

    hlin1, g1 = _tc_mm1(x, W1, dinv)
    acc1 = _sc_spmm(g1, pk3d, zeros_rows)
    hlin2, g2 = _tc_mid(acc1, hlin1, dinv, b1.reshape(1, FF), W2)
    acc2 = _sc_spmm(g2, pk3d, zeros_rows)
    hlin3, g3 = _tc_mid(acc2, hlin2, dinv, b2.reshape(1, FF), W3)
    acc3 = _sc_spmm(g3, pk3d, zeros_rows)
    return _tc_final(acc3, hlin3, dinv, b3.reshape(1, FF))
